# Initial kernel scaffold; baseline (speedup 1.0000x reference)
import jax
import jax.numpy as jnp
from jax.experimental import pallas as pl


def kernel(x, edge_index, batch, u, W1a, b1a, W1b, b1b, gn1_w, gn1_b, gn1_ms, W2a, b2a, W2b, b2b, gn2_w, gn2_b, gn2_ms, W3a, b3a, W3b, b3b, gn3_w, gn3_b, gn3_ms, Wg, bg, Wc1, bc1, Wc2, bc2):
    raise NotImplementedError("write your pallas kernel here")


# speedup vs baseline: 4.9206x; 4.9206x over previous
import functools

import jax
import jax.numpy as jnp
from jax import lax
from jax.experimental import pallas as pl
from jax.experimental.pallas import tpu as pltpu
from jax.experimental.pallas import tpu_sc as plsc

N, E, D, B = 10000, 320000, 128, 64
H = D


_NC = 2
_NS = 16
_NW = _NC * _NS
_EPW = E // _NW
_CHUNK = 80
_NIT = _EPW // _CHUNK
_RPT = 632
_NPAD = _RPT * _NS

_sc_mesh = plsc.VectorSubcoreMesh(core_axis_name="c", subcore_axis_name="s")


@functools.partial(
    pl.kernel,
    mesh=_sc_mesh,
    out_type=jax.ShapeDtypeStruct((_NC, _NPAD, D), jnp.float32),
    scratch_types=[
        pltpu.VMEM((_CHUNK,), jnp.int32),
        pltpu.VMEM((_CHUNK,), jnp.int32),
        pltpu.VMEM((_CHUNK, D), jnp.float32),
        pltpu.VMEM_SHARED((_NPAD, D), jnp.float32),
        pltpu.SemaphoreType.DMA,
    ],
)
def _agg_sc(h_hbm, src_hbm, dst_hbm, zeros_hbm, out_hbm, srcv, dstv, rows, acc, sem):
    c = lax.axis_index("c")
    s = lax.axis_index("s")
    r0 = s * _RPT
    pltpu.sync_copy(zeros_hbm.at[pl.ds(r0, _RPT)], acc.at[pl.ds(r0, _RPT)])
    plsc.subcore_barrier()
    base = (c * _NS + s) * _EPW

    def body(i, carry):
        off = base + i * _CHUNK
        pltpu.sync_copy(src_hbm.at[pl.ds(off, _CHUNK)], srcv)
        pltpu.sync_copy(dst_hbm.at[pl.ds(off, _CHUNK)], dstv)
        pltpu.async_copy(h_hbm.at[srcv], rows, sem).wait()
        pltpu.sync_copy(rows, acc.at[dstv], add=True)
        return carry

    lax.fori_loop(0, _NIT, body, 0)
    plsc.subcore_barrier()
    pltpu.sync_copy(acc.at[pl.ds(r0, _RPT)], out_hbm.at[c, pl.ds(r0, _RPT)])


_BLK = 2000
_GRID = N // _BLK


def _xdot(a, b):
    return jnp.dot(a, b, preferred_element_type=jnp.float32)


def _pt(bvec):
    return (lax.broadcasted_iota(jnp.int32, (B, bvec.shape[0]), 0)
            == bvec[None, :]).astype(jnp.float32)


def _mlp_stats_body(h_ref, a0_ref, a1_ref, bt_ref, Wa_ref, ba_ref, Wb_ref, bb_ref,
                    t_ref, s0_ref, s1_ref, s2_ref):
    i = pl.program_id(0)
    z = h_ref[...] + a0_ref[0] + a1_ref[0]
    z1 = jnp.maximum(_xdot(z, Wa_ref[...]) + ba_ref[...], 0.0)
    t = _xdot(z1, Wb_ref[...]) + bb_ref[...]
    t_ref[...] = t
    bvec = bt_ref[0, 0, :]
    PT = _pt(bvec)
    ps0 = jnp.dot(PT, jnp.ones((_BLK, H), jnp.float32),
                  preferred_element_type=jnp.float32, precision=lax.Precision.HIGHEST)
    ps1 = jnp.dot(PT, t, preferred_element_type=jnp.float32, precision=lax.Precision.HIGHEST)
    ps2 = jnp.dot(PT, t * t, preferred_element_type=jnp.float32, precision=lax.Precision.HIGHEST)

    @pl.when(i == 0)
    def _():
        s0_ref[...] = ps0
        s1_ref[...] = ps1
        s2_ref[...] = ps2

    @pl.when(i > 0)
    def _():
        s0_ref[...] += ps0
        s1_ref[...] += ps1
        s2_ref[...] += ps2


def _mlp_stats(h, agg2, batch3, Wa, ba, Wb, bb):
    return pl.pallas_call(
        _mlp_stats_body,
        grid=(_GRID,),
        in_specs=[
            pl.BlockSpec((_BLK, H), lambda i: (i, 0)),
            pl.BlockSpec((1, _BLK, D), lambda i: (0, i, 0)),
            pl.BlockSpec((1, _BLK, D), lambda i: (1, i, 0)),
            pl.BlockSpec((1, 1, _BLK), lambda i: (i, 0, 0)),
            pl.BlockSpec((D, H), lambda i: (0, 0)),
            pl.BlockSpec((1, H), lambda i: (0, 0)),
            pl.BlockSpec((H, H), lambda i: (0, 0)),
            pl.BlockSpec((1, H), lambda i: (0, 0)),
        ],
        out_specs=[
            pl.BlockSpec((_BLK, H), lambda i: (i, 0)),
            pl.BlockSpec((B, H), lambda i: (0, 0)),
            pl.BlockSpec((B, H), lambda i: (0, 0)),
            pl.BlockSpec((B, H), lambda i: (0, 0)),
        ],
        out_shape=[
            jax.ShapeDtypeStruct((N, H), jnp.float32),
            jax.ShapeDtypeStruct((B, H), jnp.float32),
            jax.ShapeDtypeStruct((B, H), jnp.float32),
            jax.ShapeDtypeStruct((B, H), jnp.float32),
        ],
    )(h, agg2, agg2, batch3, Wa, ba, Wb, bb)


def _norm_body(t_ref, s0_ref, s1_ref, s2_ref, bt_ref, ms_ref, wt_ref, bs_ref, o_ref):
    cnt = jnp.maximum(s0_ref[...], 1.0)
    s1 = s1_ref[...]
    mean = s1 / cnt
    mm = mean * ms_ref[...]
    var = (s2_ref[...] - 2.0 * mm * s1 + s0_ref[...] * mm * mm) / cnt
    inv = lax.rsqrt(var + 1e-5)
    bvec = bt_ref[0, 0, :]
    PT = _pt(bvec)
    dn = (((0,), (0,)), ((), ()))
    mean_rows = lax.dot_general(PT, mm, dn, preferred_element_type=jnp.float32, precision=lax.Precision.HIGHEST)
    inv_rows = lax.dot_general(PT, inv, dn, preferred_element_type=jnp.float32, precision=lax.Precision.HIGHEST)
    o = (t_ref[...] - mean_rows) * inv_rows * wt_ref[...] + bs_ref[...]
    o_ref[...] = jnp.maximum(o, 0.0)


def _norm(t, s0, s1, s2, batch3, ms, wt, bs):
    return pl.pallas_call(
        _norm_body,
        grid=(_GRID,),
        in_specs=[
            pl.BlockSpec((_BLK, H), lambda i: (i, 0)),
            pl.BlockSpec((B, H), lambda i: (0, 0)),
            pl.BlockSpec((B, H), lambda i: (0, 0)),
            pl.BlockSpec((B, H), lambda i: (0, 0)),
            pl.BlockSpec((1, 1, _BLK), lambda i: (i, 0, 0)),
            pl.BlockSpec((1, H), lambda i: (0, 0)),
            pl.BlockSpec((1, H), lambda i: (0, 0)),
            pl.BlockSpec((1, H), lambda i: (0, 0)),
        ],
        out_specs=pl.BlockSpec((_BLK, H), lambda i: (i, 0)),
        out_shape=jax.ShapeDtypeStruct((N, H), jnp.float32),
    )(t, s0, s1, s2, batch3, ms, wt, bs)


def _pool1_body(h1_ref, h2_ref, h3_ref, bt_ref, Wg_ref, bg_ref,
                h_ref, gmax_ref, hmax_ref, sh_ref):
    i = pl.program_id(0)
    h = h1_ref[...] + h2_ref[...] + h3_ref[...]
    h_ref[...] = h
    bvec = bt_ref[0, 0, :]
    PT = _pt(bvec)
    sh = jnp.dot(PT, h, preferred_element_type=jnp.float32, precision=lax.Precision.HIGHEST)
    gate = _xdot(h, Wg_ref[...]) + bg_ref[...]
    M = (lax.broadcasted_iota(jnp.int32, (_BLK, B), 1) == bvec[:, None]).astype(jnp.float32)
    G = gate + (M - 1.0) * 1e30
    gm = jnp.broadcast_to(jnp.max(G, axis=0)[:, None], (B, H))
    rows = []
    for b in range(B):
        mb = (bvec[:, None] == b).astype(jnp.float32)
        rows.append(jnp.max(h * mb, axis=0))
    hm = jnp.stack(rows)

    @pl.when(i == 0)
    def _():
        gmax_ref[...] = gm
        hmax_ref[...] = hm
        sh_ref[...] = sh

    @pl.when(i > 0)
    def _():
        gmax_ref[...] = jnp.maximum(gmax_ref[...], gm)
        hmax_ref[...] = jnp.maximum(hmax_ref[...], hm)
        sh_ref[...] += sh


def _pool1(h1, h2, h3, batch3, Wg, bg):
    return pl.pallas_call(
        _pool1_body,
        grid=(_GRID,),
        in_specs=[
            pl.BlockSpec((_BLK, H), lambda i: (i, 0)),
            pl.BlockSpec((_BLK, H), lambda i: (i, 0)),
            pl.BlockSpec((_BLK, H), lambda i: (i, 0)),
            pl.BlockSpec((1, 1, _BLK), lambda i: (i, 0, 0)),
            pl.BlockSpec((H, 1), lambda i: (0, 0)),
            pl.BlockSpec((1, 1), lambda i: (0, 0)),
        ],
        out_specs=[
            pl.BlockSpec((_BLK, H), lambda i: (i, 0)),
            pl.BlockSpec((B, H), lambda i: (0, 0)),
            pl.BlockSpec((B, H), lambda i: (0, 0)),
            pl.BlockSpec((B, H), lambda i: (0, 0)),
        ],
        out_shape=[
            jax.ShapeDtypeStruct((N, H), jnp.float32),
            jax.ShapeDtypeStruct((B, H), jnp.float32),
            jax.ShapeDtypeStruct((B, H), jnp.float32),
            jax.ShapeDtypeStruct((B, H), jnp.float32),
        ],
    )(h1, h2, h3, batch3, Wg, bg)


def _pool2_body(h_ref, bt_ref, Wg_ref, bg_ref, gmax_ref, sex_ref, shex_ref):
    i = pl.program_id(0)
    h = h_ref[...]
    gate = _xdot(h, Wg_ref[...]) + bg_ref[...]
    bvec = bt_ref[0, 0, :]
    PT = _pt(bvec)
    dn = (((0,), (0,)), ((), ()))
    gmax_col = gmax_ref[...][:, 0:1]
    gmax_rows = lax.dot_general(PT, gmax_col, dn, preferred_element_type=jnp.float32, precision=lax.Precision.HIGHEST)
    ex = jnp.exp(gate - gmax_rows)
    psex = jnp.dot(PT, jnp.broadcast_to(ex, (_BLK, H)),
                   preferred_element_type=jnp.float32, precision=lax.Precision.HIGHEST)
    pshex = jnp.dot(PT, h * ex, preferred_element_type=jnp.float32, precision=lax.Precision.HIGHEST)

    @pl.when(i == 0)
    def _():
        sex_ref[...] = psex
        shex_ref[...] = pshex

    @pl.when(i > 0)
    def _():
        sex_ref[...] += psex
        shex_ref[...] += pshex


def _pool2(h, batch3, Wg, bg, gmax):
    return pl.pallas_call(
        _pool2_body,
        grid=(_GRID,),
        in_specs=[
            pl.BlockSpec((_BLK, H), lambda i: (i, 0)),
            pl.BlockSpec((1, 1, _BLK), lambda i: (i, 0, 0)),
            pl.BlockSpec((H, 1), lambda i: (0, 0)),
            pl.BlockSpec((1, 1), lambda i: (0, 0)),
            pl.BlockSpec((B, H), lambda i: (0, 0)),
        ],
        out_specs=[
            pl.BlockSpec((B, H), lambda i: (0, 0)),
            pl.BlockSpec((B, H), lambda i: (0, 0)),
        ],
        out_shape=[
            jax.ShapeDtypeStruct((B, H), jnp.float32),
            jax.ShapeDtypeStruct((B, H), jnp.float32),
        ],
    )(h, batch3, Wg, bg, gmax)


def _head_body(s0_ref, sh_ref, hmax_ref, sex_ref, shex_ref, up_ref,
               Wc1_ref, bc1_ref, Wc2_ref, bc2_ref, o_ref):
    cnt = jnp.maximum(s0_ref[...], 1.0)
    att = shex_ref[...] / jnp.maximum(sex_ref[...], 1e-30)
    meanp = sh_ref[...] / cnt
    z = jnp.concatenate([att, meanp, hmax_ref[...], up_ref[...]], axis=1)
    z1 = jnp.maximum(
        _xdot(z, Wc1_ref[...]) + bc1_ref[...], 0.0)
    o_ref[...] = _xdot(z1, Wc2_ref[...]) + bc2_ref[...]


def _head(s0, sh, hmax, sex, shex, up, Wc1p, bc1, Wc2, bc2):
    return pl.pallas_call(
        _head_body,
        out_shape=jax.ShapeDtypeStruct((B, 1), jnp.float32),
    )(s0, sh, hmax, sex, shex, up, Wc1p, bc1, Wc2, bc2)


def kernel(x, edge_index, batch, u,
           W1a, b1a, W1b, b1b, gn1_w, gn1_b, gn1_ms,
           W2a, b2a, W2b, b2b, gn2_w, gn2_b, gn2_ms,
           W3a, b3a, W3b, b3b, gn3_w, gn3_b, gn3_ms,
           Wg, bg, Wc1, bc1, Wc2, bc2):
    src = edge_index[0]
    dst = edge_index[1]
    zeros = jnp.zeros((_NPAD, D), jnp.float32)
    batch3 = batch.reshape(_GRID, 1, _BLK)
    r1 = lambda v: v.reshape(1, -1)

    h = x
    hs = []
    stats = None
    for (Wa, ba, Wb, bb, ms, wt, bs) in (
            (W1a, b1a, W1b, b1b, gn1_ms, gn1_w, gn1_b),
            (W2a, b2a, W2b, b2b, gn2_ms, gn2_w, gn2_b),
            (W3a, b3a, W3b, b3b, gn3_ms, gn3_w, gn3_b)):
        agg2 = _agg_sc(h, src, dst, zeros)
        t, s0, s1, s2 = _mlp_stats(h, agg2, batch3, Wa, r1(ba), Wb, r1(bb))
        h = _norm(t, s0, s1, s2, batch3, r1(ms), r1(wt), r1(bs))
        hs.append(h)
        stats = s0

    hsum, gmax, hmax, sh = _pool1(hs[0], hs[1], hs[2], batch3, Wg, bg.reshape(1, 1))
    sex, shex = _pool2(hsum, batch3, Wg, bg.reshape(1, 1), gmax)

    up = jnp.concatenate([u, jnp.zeros((B, H - 3), jnp.float32)], axis=1)
    Wc1p = jnp.concatenate([Wc1, jnp.zeros((4 * H - (3 * H + 3), H), jnp.float32)],
                           axis=0)
    out = _head(stats, sh, hmax, sex, shex, up, Wc1p, r1(bc1), Wc2, bc2.reshape(1, 1))
    return out[:, 0]


# speedup vs baseline: 8.6185x; 1.7515x over previous
import functools

import jax
import jax.numpy as jnp
from jax import lax
from jax.experimental import pallas as pl
from jax.experimental.pallas import tpu as pltpu
from jax.experimental.pallas import tpu_sc as plsc

N, E, D, B = 10000, 320000, 128, 64
H = D


_NC = 2
_NS = 16
_NW = _NC * _NS
_EPW = E // _NW
_CHUNK = 40
_NIT = _EPW // _CHUNK
_RPT = 632
_NPAD = _RPT * _NS

_sc_mesh = plsc.VectorSubcoreMesh(core_axis_name="c", subcore_axis_name="s")

_NBUF = 5
_NIB = 10
_ROUNDS = _NIT // _NBUF


@functools.partial(
    pl.kernel,
    mesh=_sc_mesh,
    out_type=jax.ShapeDtypeStruct((_NC, _NPAD, D), jnp.float32),
    scratch_types=[
        pltpu.VMEM((_NIB, 2, _CHUNK), jnp.int32),
        pltpu.VMEM((_NBUF, _CHUNK, D), jnp.float32),
        pltpu.VMEM_SHARED((_NPAD, D), jnp.float32),
        pltpu.SemaphoreType.DMA((_NIB,)),
        pltpu.SemaphoreType.DMA((_NBUF,)),
        pltpu.SemaphoreType.DMA((_NBUF,)),
    ],
)
def _agg_sc(h_hbm, ei_hbm, zeros_hbm, out_hbm, ibuf, rbuf, acc, isem, gsem, ssem):
    c = lax.axis_index("c")
    s = lax.axis_index("s")
    r0 = s * _RPT
    pltpu.sync_copy(zeros_hbm.at[pl.ds(r0, _RPT)], acc.at[pl.ds(r0, _RPT)])
    plsc.subcore_barrier()
    wid = c * _NS + s

    def fire_idx(j, slot):
        pltpu.async_copy(ei_hbm.at[wid, j], ibuf.at[slot], isem.at[slot])

    def wait_idx(j, slot):
        pltpu.make_async_copy(ei_hbm.at[wid, j], ibuf.at[slot], isem.at[slot]).wait()

    def fire_gather(j, islot, b):
        pltpu.async_copy(h_hbm.at[ibuf.at[islot, 0]], rbuf.at[b], gsem.at[b])

    def wait_gather(j, islot, b):
        pltpu.make_async_copy(h_hbm.at[ibuf.at[islot, 0]], rbuf.at[b],
                              gsem.at[b]).wait()

    def fire_scat(j, islot, b):
        pltpu.async_copy(rbuf.at[b], acc.at[ibuf.at[islot, 1]], ssem.at[b],
                         add=True)

    def wait_scat(j, islot, b):
        pltpu.make_async_copy(rbuf.at[b], acc.at[ibuf.at[islot, 1]],
                              ssem.at[b]).wait()

    for j in range(_NBUF + 1):
        fire_idx(j, j)
    for b in range(_NBUF):
        wait_idx(b, b)
        fire_gather(b, b, b)

    def round_body(g, carry):
        for b in range(_NBUF):
            i = g * _NBUF + b
            wait_gather(i, i % _NIB, b)
            fire_scat(i, i % _NIB, b)
            if b < _NBUF - 1:
                pl.when(g < _ROUNDS - 1)(lambda: fire_idx(i + 6, (i + 6) % _NIB))
            else:
                pl.when(g < _ROUNDS - 2)(lambda: fire_idx(i + 6, (i + 6) % _NIB))
            br = (b + 2) % _NBUF

            def fire_next():
                wait_scat(i - 3, (i - 3) % _NIB, br)
                wait_idx(i + 2, (i + 2) % _NIB)
                fire_gather(i + 2, (i + 2) % _NIB, br)

            if b < _NBUF - 2:
                pl.when(g > 0)(fire_next)
            else:
                pl.when(g < _ROUNDS - 1)(fire_next)
        return carry

    lax.fori_loop(0, _ROUNDS, round_body, 0)
    for b in range(_NBUF):
        i = _NIT - _NBUF + b
        wait_scat(i, i % _NIB, b)
    plsc.subcore_barrier()
    pltpu.sync_copy(acc.at[pl.ds(r0, _RPT)], out_hbm.at[c, pl.ds(r0, _RPT)])


_BLK = 2000
_GRID = N // _BLK


def _xdot(a, b):
    return jnp.dot(a, b, preferred_element_type=jnp.float32)


def _pt(bvec):
    return (lax.broadcasted_iota(jnp.int32, (B, bvec.shape[0]), 0)
            == bvec[None, :]).astype(jnp.float32)


def _mlp_stats_body(h_ref, a0_ref, a1_ref, bt_ref, Wa_ref, ba_ref, Wb_ref, bb_ref,
                    t_ref, s0_ref, s1_ref, s2_ref):
    i = pl.program_id(0)
    z = h_ref[...] + a0_ref[0] + a1_ref[0]
    z1 = jnp.maximum(_xdot(z, Wa_ref[...]) + ba_ref[...], 0.0)
    t = _xdot(z1, Wb_ref[...]) + bb_ref[...]
    t_ref[...] = t
    bvec = bt_ref[0, 0, :]
    PT = _pt(bvec)
    ps0 = jnp.dot(PT, jnp.ones((_BLK, H), jnp.float32),
                  preferred_element_type=jnp.float32, precision=lax.Precision.HIGHEST)
    ps1 = jnp.dot(PT, t, preferred_element_type=jnp.float32, precision=lax.Precision.HIGHEST)
    ps2 = jnp.dot(PT, t * t, preferred_element_type=jnp.float32, precision=lax.Precision.HIGHEST)

    @pl.when(i == 0)
    def _():
        s0_ref[...] = ps0
        s1_ref[...] = ps1
        s2_ref[...] = ps2

    @pl.when(i > 0)
    def _():
        s0_ref[...] += ps0
        s1_ref[...] += ps1
        s2_ref[...] += ps2


def _mlp_stats(h, agg2, batch3, Wa, ba, Wb, bb):
    return pl.pallas_call(
        _mlp_stats_body,
        grid=(_GRID,),
        in_specs=[
            pl.BlockSpec((_BLK, H), lambda i: (i, 0)),
            pl.BlockSpec((1, _BLK, D), lambda i: (0, i, 0)),
            pl.BlockSpec((1, _BLK, D), lambda i: (1, i, 0)),
            pl.BlockSpec((1, 1, _BLK), lambda i: (i, 0, 0)),
            pl.BlockSpec((D, H), lambda i: (0, 0)),
            pl.BlockSpec((1, H), lambda i: (0, 0)),
            pl.BlockSpec((H, H), lambda i: (0, 0)),
            pl.BlockSpec((1, H), lambda i: (0, 0)),
        ],
        out_specs=[
            pl.BlockSpec((_BLK, H), lambda i: (i, 0)),
            pl.BlockSpec((B, H), lambda i: (0, 0)),
            pl.BlockSpec((B, H), lambda i: (0, 0)),
            pl.BlockSpec((B, H), lambda i: (0, 0)),
        ],
        out_shape=[
            jax.ShapeDtypeStruct((N, H), jnp.float32),
            jax.ShapeDtypeStruct((B, H), jnp.float32),
            jax.ShapeDtypeStruct((B, H), jnp.float32),
            jax.ShapeDtypeStruct((B, H), jnp.float32),
        ],
    )(h, agg2, agg2, batch3, Wa, ba, Wb, bb)


def _norm_body(t_ref, s0_ref, s1_ref, s2_ref, bt_ref, ms_ref, wt_ref, bs_ref, o_ref):
    cnt = jnp.maximum(s0_ref[...], 1.0)
    s1 = s1_ref[...]
    mean = s1 / cnt
    mm = mean * ms_ref[...]
    var = (s2_ref[...] - 2.0 * mm * s1 + s0_ref[...] * mm * mm) / cnt
    inv = lax.rsqrt(var + 1e-5)
    bvec = bt_ref[0, 0, :]
    PT = _pt(bvec)
    dn = (((0,), (0,)), ((), ()))
    mean_rows = lax.dot_general(PT, mm, dn, preferred_element_type=jnp.float32, precision=lax.Precision.HIGHEST)
    inv_rows = lax.dot_general(PT, inv, dn, preferred_element_type=jnp.float32, precision=lax.Precision.HIGHEST)
    o = (t_ref[...] - mean_rows) * inv_rows * wt_ref[...] + bs_ref[...]
    o_ref[...] = jnp.maximum(o, 0.0)


def _norm(t, s0, s1, s2, batch3, ms, wt, bs):
    return pl.pallas_call(
        _norm_body,
        grid=(_GRID,),
        in_specs=[
            pl.BlockSpec((_BLK, H), lambda i: (i, 0)),
            pl.BlockSpec((B, H), lambda i: (0, 0)),
            pl.BlockSpec((B, H), lambda i: (0, 0)),
            pl.BlockSpec((B, H), lambda i: (0, 0)),
            pl.BlockSpec((1, 1, _BLK), lambda i: (i, 0, 0)),
            pl.BlockSpec((1, H), lambda i: (0, 0)),
            pl.BlockSpec((1, H), lambda i: (0, 0)),
            pl.BlockSpec((1, H), lambda i: (0, 0)),
        ],
        out_specs=pl.BlockSpec((_BLK, H), lambda i: (i, 0)),
        out_shape=jax.ShapeDtypeStruct((N, H), jnp.float32),
    )(t, s0, s1, s2, batch3, ms, wt, bs)


def _pool1_body(h1_ref, h2_ref, h3_ref, bt_ref, Wg_ref, bg_ref,
                h_ref, gmax_ref, hmax_ref, sh_ref):
    i = pl.program_id(0)
    h = h1_ref[...] + h2_ref[...] + h3_ref[...]
    h_ref[...] = h
    bvec = bt_ref[0, 0, :]
    PT = _pt(bvec)
    sh = jnp.dot(PT, h, preferred_element_type=jnp.float32, precision=lax.Precision.HIGHEST)
    gate = _xdot(h, Wg_ref[...]) + bg_ref[...]
    M = (lax.broadcasted_iota(jnp.int32, (_BLK, B), 1) == bvec[:, None]).astype(jnp.float32)
    G = gate + (M - 1.0) * 1e30
    gm = jnp.broadcast_to(jnp.max(G, axis=0)[:, None], (B, H))
    rows = []
    for b in range(B):
        mb = (bvec[:, None] == b).astype(jnp.float32)
        rows.append(jnp.max(h * mb, axis=0))
    hm = jnp.stack(rows)

    @pl.when(i == 0)
    def _():
        gmax_ref[...] = gm
        hmax_ref[...] = hm
        sh_ref[...] = sh

    @pl.when(i > 0)
    def _():
        gmax_ref[...] = jnp.maximum(gmax_ref[...], gm)
        hmax_ref[...] = jnp.maximum(hmax_ref[...], hm)
        sh_ref[...] += sh


def _pool1(h1, h2, h3, batch3, Wg, bg):
    return pl.pallas_call(
        _pool1_body,
        grid=(_GRID,),
        in_specs=[
            pl.BlockSpec((_BLK, H), lambda i: (i, 0)),
            pl.BlockSpec((_BLK, H), lambda i: (i, 0)),
            pl.BlockSpec((_BLK, H), lambda i: (i, 0)),
            pl.BlockSpec((1, 1, _BLK), lambda i: (i, 0, 0)),
            pl.BlockSpec((H, 1), lambda i: (0, 0)),
            pl.BlockSpec((1, 1), lambda i: (0, 0)),
        ],
        out_specs=[
            pl.BlockSpec((_BLK, H), lambda i: (i, 0)),
            pl.BlockSpec((B, H), lambda i: (0, 0)),
            pl.BlockSpec((B, H), lambda i: (0, 0)),
            pl.BlockSpec((B, H), lambda i: (0, 0)),
        ],
        out_shape=[
            jax.ShapeDtypeStruct((N, H), jnp.float32),
            jax.ShapeDtypeStruct((B, H), jnp.float32),
            jax.ShapeDtypeStruct((B, H), jnp.float32),
            jax.ShapeDtypeStruct((B, H), jnp.float32),
        ],
    )(h1, h2, h3, batch3, Wg, bg)


def _pool2_body(h_ref, bt_ref, Wg_ref, bg_ref, gmax_ref, sex_ref, shex_ref):
    i = pl.program_id(0)
    h = h_ref[...]
    gate = _xdot(h, Wg_ref[...]) + bg_ref[...]
    bvec = bt_ref[0, 0, :]
    PT = _pt(bvec)
    dn = (((0,), (0,)), ((), ()))
    gmax_col = gmax_ref[...][:, 0:1]
    gmax_rows = lax.dot_general(PT, gmax_col, dn, preferred_element_type=jnp.float32, precision=lax.Precision.HIGHEST)
    ex = jnp.exp(gate - gmax_rows)
    psex = jnp.dot(PT, jnp.broadcast_to(ex, (_BLK, H)),
                   preferred_element_type=jnp.float32, precision=lax.Precision.HIGHEST)
    pshex = jnp.dot(PT, h * ex, preferred_element_type=jnp.float32, precision=lax.Precision.HIGHEST)

    @pl.when(i == 0)
    def _():
        sex_ref[...] = psex
        shex_ref[...] = pshex

    @pl.when(i > 0)
    def _():
        sex_ref[...] += psex
        shex_ref[...] += pshex


def _pool2(h, batch3, Wg, bg, gmax):
    return pl.pallas_call(
        _pool2_body,
        grid=(_GRID,),
        in_specs=[
            pl.BlockSpec((_BLK, H), lambda i: (i, 0)),
            pl.BlockSpec((1, 1, _BLK), lambda i: (i, 0, 0)),
            pl.BlockSpec((H, 1), lambda i: (0, 0)),
            pl.BlockSpec((1, 1), lambda i: (0, 0)),
            pl.BlockSpec((B, H), lambda i: (0, 0)),
        ],
        out_specs=[
            pl.BlockSpec((B, H), lambda i: (0, 0)),
            pl.BlockSpec((B, H), lambda i: (0, 0)),
        ],
        out_shape=[
            jax.ShapeDtypeStruct((B, H), jnp.float32),
            jax.ShapeDtypeStruct((B, H), jnp.float32),
        ],
    )(h, batch3, Wg, bg, gmax)


def _head_body(s0_ref, sh_ref, hmax_ref, sex_ref, shex_ref, up_ref,
               Wc1_ref, bc1_ref, Wc2_ref, bc2_ref, o_ref):
    cnt = jnp.maximum(s0_ref[...], 1.0)
    att = shex_ref[...] / jnp.maximum(sex_ref[...], 1e-30)
    meanp = sh_ref[...] / cnt
    z = jnp.concatenate([att, meanp, hmax_ref[...], up_ref[...]], axis=1)
    z1 = jnp.maximum(
        _xdot(z, Wc1_ref[...]) + bc1_ref[...], 0.0)
    o_ref[...] = _xdot(z1, Wc2_ref[...]) + bc2_ref[...]


def _head(s0, sh, hmax, sex, shex, up, Wc1p, bc1, Wc2, bc2):
    return pl.pallas_call(
        _head_body,
        out_shape=jax.ShapeDtypeStruct((B, 1), jnp.float32),
    )(s0, sh, hmax, sex, shex, up, Wc1p, bc1, Wc2, bc2)


def kernel(x, edge_index, batch, u,
           W1a, b1a, W1b, b1b, gn1_w, gn1_b, gn1_ms,
           W2a, b2a, W2b, b2b, gn2_w, gn2_b, gn2_ms,
           W3a, b3a, W3b, b3b, gn3_w, gn3_b, gn3_ms,
           Wg, bg, Wc1, bc1, Wc2, bc2):
    ei = jnp.stack([edge_index[0].reshape(_NW, _NIT, _CHUNK),
                    edge_index[1].reshape(_NW, _NIT, _CHUNK)], axis=2)
    zeros = jnp.zeros((_NPAD, D), jnp.float32)
    batch3 = batch.reshape(_GRID, 1, _BLK)
    r1 = lambda v: v.reshape(1, -1)

    h = x
    hs = []
    stats = None
    for (Wa, ba, Wb, bb, ms, wt, bs) in (
            (W1a, b1a, W1b, b1b, gn1_ms, gn1_w, gn1_b),
            (W2a, b2a, W2b, b2b, gn2_ms, gn2_w, gn2_b),
            (W3a, b3a, W3b, b3b, gn3_ms, gn3_w, gn3_b)):
        agg2 = _agg_sc(h, ei, zeros)
        t, s0, s1, s2 = _mlp_stats(h, agg2, batch3, Wa, r1(ba), Wb, r1(bb))
        h = _norm(t, s0, s1, s2, batch3, r1(ms), r1(wt), r1(bs))
        hs.append(h)
        stats = s0

    hsum, gmax, hmax, sh = _pool1(hs[0], hs[1], hs[2], batch3, Wg, bg.reshape(1, 1))
    sex, shex = _pool2(hsum, batch3, Wg, bg.reshape(1, 1), gmax)

    up = jnp.concatenate([u, jnp.zeros((B, H - 3), jnp.float32)], axis=1)
    Wc1p = jnp.concatenate([Wc1, jnp.zeros((4 * H - (3 * H + 3), H), jnp.float32)],
                           axis=0)
    out = _head(stats, sh, hmax, sex, shex, up, Wc1p, r1(bc1), Wc2, bc2.reshape(1, 1))
    return out[:, 0]


# speedup vs baseline: 8.6593x; 1.0047x over previous
import functools

import jax
import jax.numpy as jnp
from jax import lax
from jax.experimental import pallas as pl
from jax.experimental.pallas import tpu as pltpu
from jax.experimental.pallas import tpu_sc as plsc

N, E, D, B = 10000, 320000, 128, 64
H = D


_NC = 2
_NS = 16
_NW = _NC * _NS
_EPW = E // _NW
_CHUNK = 40
_NIT = _EPW // _CHUNK
_RPT = 632
_NPAD = _RPT * _NS

_sc_mesh = plsc.VectorSubcoreMesh(core_axis_name="c", subcore_axis_name="s")

_NBUF = 5
_NIB = 10
_ROUNDS = _NIT // _NBUF


@functools.partial(
    pl.kernel,
    mesh=_sc_mesh,
    out_type=jax.ShapeDtypeStruct((_NC, _NPAD, D), jnp.float32),
    scratch_types=[
        pltpu.VMEM((_NIB, 2, _CHUNK), jnp.int32),
        pltpu.VMEM((_NBUF, _CHUNK, D), jnp.float32),
        pltpu.VMEM_SHARED((_NPAD, D), jnp.float32),
        pltpu.SemaphoreType.DMA((_NIB,)),
        pltpu.SemaphoreType.DMA((_NBUF,)),
        pltpu.SemaphoreType.DMA((_NBUF,)),
    ],
)
def _agg_sc(h_hbm, ei_hbm, zeros_hbm, out_hbm, ibuf, rbuf, acc, isem, gsem, ssem):
    c = lax.axis_index("c")
    s = lax.axis_index("s")
    r0 = s * _RPT
    pltpu.sync_copy(zeros_hbm.at[pl.ds(r0, _RPT)], acc.at[pl.ds(r0, _RPT)])
    plsc.subcore_barrier()
    wid = c * _NS + s

    def fire_idx(j, slot):
        pltpu.async_copy(ei_hbm.at[wid, j], ibuf.at[slot], isem.at[slot])

    def wait_idx(j, slot):
        pltpu.make_async_copy(ei_hbm.at[wid, j], ibuf.at[slot], isem.at[slot]).wait()

    def fire_gather(islot, b):
        pltpu.async_copy(h_hbm.at[ibuf.at[islot, 0]], rbuf.at[b], gsem.at[b])

    def wait_gather(islot, b):
        pltpu.make_async_copy(h_hbm.at[ibuf.at[islot, 0]], rbuf.at[b],
                              gsem.at[b]).wait()

    def fire_scat(islot, b):
        pltpu.async_copy(rbuf.at[b], acc.at[ibuf.at[islot, 1]], ssem.at[b],
                         add=True)

    def wait_scat(islot, b):
        pltpu.make_async_copy(rbuf.at[b], acc.at[ibuf.at[islot, 1]],
                              ssem.at[b]).wait()

    for j in range(_NBUF + 1):
        fire_idx(j, j)
    for b in range(_NBUF):
        wait_idx(b, b)
        fire_gather(b, b)

    def round_body(g, carry):
        for b in range(_NBUF):
            i = g * _NBUF + b
            wait_gather(i % _NIB, b)
            fire_scat(i % _NIB, b)
            if b < _NBUF - 1:
                pl.when(g < _ROUNDS - 1)(lambda: fire_idx(i + 6, (i + 6) % _NIB))
            else:
                pl.when(g < _ROUNDS - 2)(lambda: fire_idx(i + 6, (i + 6) % _NIB))
            br = (b + 2) % _NBUF

            def fire_next():
                wait_scat((i - 3) % _NIB, br)
                wait_idx(i + 2, (i + 2) % _NIB)
                fire_gather((i + 2) % _NIB, br)

            if b < _NBUF - 2:
                pl.when(g > 0)(fire_next)
            else:
                pl.when(g < _ROUNDS - 1)(fire_next)
        return carry

    lax.fori_loop(0, _ROUNDS, round_body, 0)
    for b in range(_NBUF):
        i = _NIT - _NBUF + b
        wait_scat(i % _NIB, b)
    plsc.subcore_barrier()
    pltpu.sync_copy(acc.at[pl.ds(r0, _RPT)], out_hbm.at[c, pl.ds(r0, _RPT)])


_BLK = 2000
_GRID = N // _BLK
_HI = lax.Precision.HIGHEST


def _xdot(a, b):
    return jnp.dot(a, b, preferred_element_type=jnp.float32)


def _pt(bvec):
    return (lax.broadcasted_iota(jnp.int32, (B, bvec.shape[0]), 0)
            == bvec[None, :]).astype(jnp.float32)


def _hdot(a, b):
    return jnp.dot(a, b, preferred_element_type=jnp.float32, precision=_HI)


_DN0 = (((0,), (0,)), ((), ()))


def _norm_from_stats(ts_ref, j, bt_ref, ms_ref, wt_ref, bs_ref,
                     s0_ref, s1_ref, s2_ref):
    cnt = jnp.maximum(s0_ref[...], 1.0)
    s1 = s1_ref[...]
    mean = s1 / cnt
    mm = mean * ms_ref[...]
    var = (s2_ref[...] - 2.0 * mm * s1 + s0_ref[...] * mm * mm) / cnt
    inv = lax.rsqrt(var + 1e-5)
    bvec = bt_ref[0, 0, :]
    PT = _pt(bvec)
    mean_rows = lax.dot_general(PT, mm, _DN0, preferred_element_type=jnp.float32,
                                precision=_HI)
    inv_rows = lax.dot_general(PT, inv, _DN0, preferred_element_type=jnp.float32,
                               precision=_HI)
    t = ts_ref[pl.ds(j * _BLK, _BLK), :]
    o = (t - mean_rows) * inv_rows * wt_ref[...] + bs_ref[...]
    return jnp.maximum(o, 0.0)


def _mlp_stats_phase(h_ref, a0_ref, a1_ref, bt_ref, Wa_ref, ba_ref, Wb_ref,
                     bb_ref, ts_ref, s0_ref, s1_ref, s2_ref, i, j):
    z = h_ref[...] + a0_ref[0] + a1_ref[0]
    z1 = jnp.maximum(_xdot(z, Wa_ref[...]) + ba_ref[...], 0.0)
    t = _xdot(z1, Wb_ref[...]) + bb_ref[...]
    ts_ref[pl.ds(j * _BLK, _BLK), :] = t
    bvec = bt_ref[0, 0, :]
    PT = _pt(bvec)
    ps0 = _hdot(PT, jnp.ones((_BLK, H), jnp.float32))
    ps1 = _hdot(PT, t)
    ps2 = _hdot(PT, t * t)

    @pl.when(i == 0)
    def _():
        s0_ref[...] = ps0
        s1_ref[...] = ps1
        s2_ref[...] = ps2

    @pl.when(i > 0)
    def _():
        s0_ref[...] += ps0
        s1_ref[...] += ps1
        s2_ref[...] += ps2


def _layer_body(h_ref, a0_ref, a1_ref, bt_ref, Wa_ref, ba_ref, Wb_ref, bb_ref,
                ms_ref, wt_ref, bs_ref, o_ref, ts_ref, s0_ref, s1_ref, s2_ref):
    i = pl.program_id(0)
    j = i % _GRID

    @pl.when(i < _GRID)
    def _():
        _mlp_stats_phase(h_ref, a0_ref, a1_ref, bt_ref, Wa_ref, ba_ref, Wb_ref,
                         bb_ref, ts_ref, s0_ref, s1_ref, s2_ref, i, j)

    @pl.when(i >= _GRID)
    def _():
        o_ref[...] = _norm_from_stats(ts_ref, j, bt_ref, ms_ref, wt_ref, bs_ref,
                                      s0_ref, s1_ref, s2_ref)


def _w1(i):
    return (jnp.where(i < _GRID, i, 0), 0)


def _w2(i):
    return (jnp.where(i < _GRID, 0, i % _GRID), 0)


def _jj(i):
    return (i % _GRID, 0)


def _jj3(i):
    return (i % _GRID, 0, 0)


def _c2(i):
    return (0, 0)


_WSPEC = pl.BlockSpec((D, H), _c2)
_BSPEC = pl.BlockSpec((1, H), _c2)


def _layer_tc(h, agg2, batch3, Wa, ba, Wb, bb, ms, wt, bs):
    return pl.pallas_call(
        _layer_body,
        grid=(2 * _GRID,),
        in_specs=[
            pl.BlockSpec((_BLK, H), _w1),
            pl.BlockSpec((1, _BLK, D), lambda i: (0, jnp.where(i < _GRID, i, 0), 0)),
            pl.BlockSpec((1, _BLK, D), lambda i: (1, jnp.where(i < _GRID, i, 0), 0)),
            pl.BlockSpec((1, 1, _BLK), _jj3),
            _WSPEC, _BSPEC, _WSPEC, _BSPEC,
            _BSPEC, _BSPEC, _BSPEC,
        ],
        out_specs=pl.BlockSpec((_BLK, H), _w2),
        out_shape=jax.ShapeDtypeStruct((N, H), jnp.float32),
        scratch_shapes=[
            pltpu.VMEM((N, H), jnp.float32),
            pltpu.VMEM((B, H), jnp.float32),
            pltpu.VMEM((B, H), jnp.float32),
            pltpu.VMEM((B, H), jnp.float32),
        ],
    )(h, agg2, agg2, batch3, Wa, ba, Wb, bb, ms, wt, bs)


def _layer3_pool_body(h_ref, a0_ref, a1_ref, bt_ref, Wa_ref, ba_ref, Wb_ref,
                      bb_ref, ms_ref, wt_ref, bs_ref, h1_ref, Wg_ref,
                      bg_ref, ho_ref, s0o_ref, gmax_ref, hmax_ref, sh_ref,
                      ts_ref, s0_ref, s1_ref, s2_ref):
    i = pl.program_id(0)
    j = i % _GRID

    @pl.when(i < _GRID)
    def _():
        _mlp_stats_phase(h_ref, a0_ref, a1_ref, bt_ref, Wa_ref, ba_ref, Wb_ref,
                         bb_ref, ts_ref, s0_ref, s1_ref, s2_ref, i, j)

    @pl.when(i >= _GRID)
    def _():
        h3 = _norm_from_stats(ts_ref, j, bt_ref, ms_ref, wt_ref, bs_ref,
                              s0_ref, s1_ref, s2_ref)
        h = h1_ref[...] + h_ref[...] + h3
        ho_ref[...] = h
        bvec = bt_ref[0, 0, :]
        PT = _pt(bvec)
        sh = _hdot(PT, h)
        gate = _xdot(h, Wg_ref[...]) + bg_ref[...]
        M = (lax.broadcasted_iota(jnp.int32, (_BLK, B), 1)
             == bvec[:, None]).astype(jnp.float32)
        G = gate + (M - 1.0) * 1e30
        gm = jnp.broadcast_to(jnp.max(G, axis=0)[:, None], (B, H))
        rows = []
        for b in range(B):
            mb = (bvec[:, None] == b).astype(jnp.float32)
            rows.append(jnp.max(h * mb, axis=0))
        hm = jnp.stack(rows)

        @pl.when(i == _GRID)
        def _():
            s0o_ref[...] = s0_ref[...]
            gmax_ref[...] = gm
            hmax_ref[...] = hm
            sh_ref[...] = sh

        @pl.when(i > _GRID)
        def _():
            gmax_ref[...] = jnp.maximum(gmax_ref[...], gm)
            hmax_ref[...] = jnp.maximum(hmax_ref[...], hm)
            sh_ref[...] += sh


def _layer3_pool(h, agg2, batch3, Wa, ba, Wb, bb, ms, wt, bs, h1, Wg, bg):
    bh = jax.ShapeDtypeStruct((B, H), jnp.float32)
    return pl.pallas_call(
        _layer3_pool_body,
        grid=(2 * _GRID,),
        in_specs=[
            pl.BlockSpec((_BLK, H), _jj),
            pl.BlockSpec((1, _BLK, D), lambda i: (0, jnp.where(i < _GRID, i, 0), 0)),
            pl.BlockSpec((1, _BLK, D), lambda i: (1, jnp.where(i < _GRID, i, 0), 0)),
            pl.BlockSpec((1, 1, _BLK), _jj3),
            _WSPEC, _BSPEC, _WSPEC, _BSPEC,
            _BSPEC, _BSPEC, _BSPEC,
            pl.BlockSpec((_BLK, H), _w2),
            pl.BlockSpec((H, 1), _c2),
            pl.BlockSpec((1, 1), _c2),
        ],
        out_specs=[
            pl.BlockSpec((_BLK, H), _w2),
            pl.BlockSpec((B, H), _c2),
            pl.BlockSpec((B, H), _c2),
            pl.BlockSpec((B, H), _c2),
            pl.BlockSpec((B, H), _c2),
        ],
        out_shape=[jax.ShapeDtypeStruct((N, H), jnp.float32), bh, bh, bh, bh],
        scratch_shapes=[
            pltpu.VMEM((N, H), jnp.float32),
            pltpu.VMEM((B, H), jnp.float32),
            pltpu.VMEM((B, H), jnp.float32),
            pltpu.VMEM((B, H), jnp.float32),
        ],
    )(h, agg2, agg2, batch3, Wa, ba, Wb, bb, ms, wt, bs, h1, Wg, bg)


def _pool2_head_body(h_ref, bt_ref, Wg_ref, bg_ref, gmax_ref, s0_ref, sh_ref,
                     hmax_ref, up_ref, Wc1_ref, bc1_ref, Wc2_ref, bc2_ref,
                     o_ref, sex_ref, shex_ref):
    i = pl.program_id(0)

    @pl.when(i < _GRID)
    def _():
        h = h_ref[...]
        gate = _xdot(h, Wg_ref[...]) + bg_ref[...]
        bvec = bt_ref[0, 0, :]
        PT = _pt(bvec)
        gmax_col = gmax_ref[...][:, 0:1]
        gmax_rows = lax.dot_general(PT, gmax_col, _DN0,
                                    preferred_element_type=jnp.float32,
                                    precision=_HI)
        ex = jnp.exp(gate - gmax_rows)
        psex = _hdot(PT, jnp.broadcast_to(ex, (_BLK, H)))
        pshex = _hdot(PT, h * ex)

        @pl.when(i == 0)
        def _():
            sex_ref[...] = psex
            shex_ref[...] = pshex

        @pl.when(i > 0)
        def _():
            sex_ref[...] += psex
            shex_ref[...] += pshex

    @pl.when(i == _GRID)
    def _():
        cnt = jnp.maximum(s0_ref[...], 1.0)
        att = shex_ref[...] / jnp.maximum(sex_ref[...], 1e-30)
        meanp = sh_ref[...] / cnt
        z = jnp.concatenate([att, meanp, hmax_ref[...], up_ref[...]], axis=1)
        z1 = jnp.maximum(_xdot(z, Wc1_ref[...]) + bc1_ref[...], 0.0)
        o_ref[...] = _xdot(z1, Wc2_ref[...]) + bc2_ref[...]


def _pool2_head(h, batch3, Wg, bg, gmax, s0, sh, hmax, up, Wc1p, bc1, Wc2, bc2):
    bhspec = pl.BlockSpec((B, H), _c2)
    return pl.pallas_call(
        _pool2_head_body,
        grid=(_GRID + 1,),
        in_specs=[
            pl.BlockSpec((_BLK, H), lambda i: (jnp.where(i < _GRID, i, 0), 0)),
            pl.BlockSpec((1, 1, _BLK), lambda i: (jnp.where(i < _GRID, i, 0), 0, 0)),
            pl.BlockSpec((H, 1), _c2),
            pl.BlockSpec((1, 1), _c2),
            bhspec, bhspec, bhspec, bhspec,
            pl.BlockSpec((B, H), _c2),
            pl.BlockSpec((4 * H, H), _c2),
            _BSPEC,
            pl.BlockSpec((H, 1), _c2),
            pl.BlockSpec((1, 1), _c2),
        ],
        out_specs=pl.BlockSpec((B, 1), _c2),
        out_shape=jax.ShapeDtypeStruct((B, 1), jnp.float32),
        scratch_shapes=[
            pltpu.VMEM((B, H), jnp.float32),
            pltpu.VMEM((B, H), jnp.float32),
        ],
    )(h, batch3, Wg, bg, gmax, s0, sh, hmax, up, Wc1p, bc1, Wc2, bc2)


def kernel(x, edge_index, batch, u,
           W1a, b1a, W1b, b1b, gn1_w, gn1_b, gn1_ms,
           W2a, b2a, W2b, b2b, gn2_w, gn2_b, gn2_ms,
           W3a, b3a, W3b, b3b, gn3_w, gn3_b, gn3_ms,
           Wg, bg, Wc1, bc1, Wc2, bc2):
    ei = jnp.stack([edge_index[0].reshape(_NW, _NIT, _CHUNK),
                    edge_index[1].reshape(_NW, _NIT, _CHUNK)], axis=2)
    zeros = jnp.zeros((_NPAD, D), jnp.float32)
    batch3 = batch.reshape(_GRID, 1, _BLK)
    r1 = lambda v: v.reshape(1, -1)

    agg1 = _agg_sc(x, ei, zeros)
    h1 = _layer_tc(x, agg1, batch3, W1a, r1(b1a), W1b, r1(b1b),
                   r1(gn1_ms), r1(gn1_w), r1(gn1_b))
    agg2 = _agg_sc(h1, ei, zeros)
    h2 = _layer_tc(h1, agg2, batch3, W2a, r1(b2a), W2b, r1(b2b),
                   r1(gn2_ms), r1(gn2_w), r1(gn2_b))
    agg3 = _agg_sc(h2, ei, zeros)
    hsum, s0, gmax, hmax, sh = _layer3_pool(
        h2, agg3, batch3, W3a, r1(b3a), W3b, r1(b3b),
        r1(gn3_ms), r1(gn3_w), r1(gn3_b), h1, Wg, bg.reshape(1, 1))

    up = jnp.concatenate([u, jnp.zeros((B, H - 3), jnp.float32)], axis=1)
    Wc1p = jnp.concatenate([Wc1, jnp.zeros((4 * H - (3 * H + 3), H), jnp.float32)],
                           axis=0)
    out = _pool2_head(hsum, batch3, Wg, bg.reshape(1, 1), gmax, s0, sh, hmax,
                      up, Wc1p, r1(bc1), Wc2, bc2.reshape(1, 1))
    return out[:, 0]


# speedup vs baseline: 8.6728x; 1.0016x over previous
import functools

import jax
import jax.numpy as jnp
from jax import lax
from jax.experimental import pallas as pl
from jax.experimental.pallas import tpu as pltpu
from jax.experimental.pallas import tpu_sc as plsc

N, E, D, B = 10000, 320000, 128, 64
H = D


_NC = 2
_NS = 16
_NW = _NC * _NS
_EPW = E // _NW
_CHUNK = 40
_NIT = _EPW // _CHUNK
_RPT = 632
_NPAD = _RPT * _NS

_sc_mesh = plsc.VectorSubcoreMesh(core_axis_name="c", subcore_axis_name="s")

_NBUF = 5
_KAHEAD = 2
_NIB = 10
_ROUNDS = _NIT // _NBUF


@functools.partial(
    pl.kernel,
    mesh=_sc_mesh,
    out_type=jax.ShapeDtypeStruct((_NC, _NPAD, D), jnp.float32),
    scratch_types=[
        pltpu.VMEM((_NIB, 2, _CHUNK), jnp.int32),
        pltpu.VMEM((_NBUF, _CHUNK, D), jnp.float32),
        pltpu.VMEM_SHARED((_NPAD, D), jnp.float32),
        pltpu.SemaphoreType.DMA((_NIB,)),
        pltpu.SemaphoreType.DMA((_NBUF,)),
        pltpu.SemaphoreType.DMA((_NBUF,)),
    ],
)
def _agg_sc(h_hbm, ei_hbm, zeros_hbm, out_hbm, ibuf, rbuf, acc, isem, gsem, ssem):
    c = lax.axis_index("c")
    s = lax.axis_index("s")
    r0 = s * _RPT
    pltpu.sync_copy(zeros_hbm.at[pl.ds(r0, _RPT)], acc.at[pl.ds(r0, _RPT)])
    plsc.subcore_barrier()
    wid = c * _NS + s

    def fire_idx(j, slot):
        pltpu.async_copy(ei_hbm.at[wid, j], ibuf.at[slot], isem.at[slot])

    def wait_idx(j, slot):
        pltpu.make_async_copy(ei_hbm.at[wid, j], ibuf.at[slot], isem.at[slot]).wait()

    def fire_gather(islot, b):
        pltpu.async_copy(h_hbm.at[ibuf.at[islot, 0]], rbuf.at[b], gsem.at[b])

    def wait_gather(islot, b):
        pltpu.make_async_copy(h_hbm.at[ibuf.at[islot, 0]], rbuf.at[b],
                              gsem.at[b]).wait()

    def fire_scat(islot, b):
        pltpu.async_copy(rbuf.at[b], acc.at[ibuf.at[islot, 1]], ssem.at[b],
                         add=True)

    def wait_scat(islot, b):
        pltpu.make_async_copy(rbuf.at[b], acc.at[ibuf.at[islot, 1]],
                              ssem.at[b]).wait()

    for j in range(_NBUF + 1):
        fire_idx(j, j)
    for b in range(_NBUF):
        wait_idx(b, b)
        fire_gather(b, b)

    def round_body(g, carry):
        for b in range(_NBUF):
            i = g * _NBUF + b
            wait_gather(i % _NIB, b)
            fire_scat(i % _NIB, b)
            if b < _NBUF - 1:
                pl.when(g < _ROUNDS - 1)(lambda: fire_idx(i + 6, (i + 6) % _NIB))
            else:
                pl.when(g < _ROUNDS - 2)(lambda: fire_idx(i + 6, (i + 6) % _NIB))
            br = (b + _KAHEAD) % _NBUF

            def fire_next():
                wait_scat((i - (_NBUF - _KAHEAD)) % _NIB, br)
                wait_idx(i + _KAHEAD, (i + _KAHEAD) % _NIB)
                fire_gather((i + _KAHEAD) % _NIB, br)

            if b < _NBUF - _KAHEAD:
                pl.when(g > 0)(fire_next)
            else:
                pl.when(g < _ROUNDS - 1)(fire_next)
        return carry

    lax.fori_loop(0, _ROUNDS, round_body, 0)
    for b in range(_NBUF):
        i = _NIT - _NBUF + b
        wait_scat(i % _NIB, b)
    plsc.subcore_barrier()
    pltpu.sync_copy(acc.at[pl.ds(r0, _RPT)], out_hbm.at[c, pl.ds(r0, _RPT)])


_BLK = 2000
_GRID = N // _BLK
_HI = lax.Precision.HIGHEST


def _xdot(a, b):
    return jnp.dot(a, b, preferred_element_type=jnp.float32)


def _pt(bvec):
    return (lax.broadcasted_iota(jnp.int32, (B, bvec.shape[0]), 0)
            == bvec[None, :]).astype(jnp.float32)


def _hdot(a, b):
    return jnp.dot(a, b, preferred_element_type=jnp.float32, precision=_HI)


_DN0 = (((0,), (0,)), ((), ()))


def _norm_from_stats(ts_ref, j, bt_ref, ms_ref, wt_ref, bs_ref,
                     s0_ref, s1_ref, s2_ref):
    cnt = jnp.maximum(s0_ref[...], 1.0)
    s1 = s1_ref[...]
    mean = s1 / cnt
    mm = mean * ms_ref[...]
    var = (s2_ref[...] - 2.0 * mm * s1 + s0_ref[...] * mm * mm) / cnt
    inv = lax.rsqrt(var + 1e-5)
    bvec = bt_ref[0, 0, :]
    PT = _pt(bvec)
    mean_rows = lax.dot_general(PT, mm, _DN0, preferred_element_type=jnp.float32,
                                precision=_HI)
    inv_rows = lax.dot_general(PT, inv, _DN0, preferred_element_type=jnp.float32,
                               precision=_HI)
    t = ts_ref[pl.ds(j * _BLK, _BLK), :]
    o = (t - mean_rows) * inv_rows * wt_ref[...] + bs_ref[...]
    return jnp.maximum(o, 0.0)


def _mlp_stats_phase(h_ref, a0_ref, a1_ref, bt_ref, Wa_ref, ba_ref, Wb_ref,
                     bb_ref, ts_ref, s0_ref, s1_ref, s2_ref, i, j):
    z = h_ref[...] + a0_ref[0] + a1_ref[0]
    z1 = jnp.maximum(_xdot(z, Wa_ref[...]) + ba_ref[...], 0.0)
    t = _xdot(z1, Wb_ref[...]) + bb_ref[...]
    ts_ref[pl.ds(j * _BLK, _BLK), :] = t
    bvec = bt_ref[0, 0, :]
    PT = _pt(bvec)
    ps0 = _hdot(PT, jnp.ones((_BLK, H), jnp.float32))
    ps1 = _hdot(PT, t)
    ps2 = _hdot(PT, t * t)

    @pl.when(i == 0)
    def _():
        s0_ref[...] = ps0
        s1_ref[...] = ps1
        s2_ref[...] = ps2

    @pl.when(i > 0)
    def _():
        s0_ref[...] += ps0
        s1_ref[...] += ps1
        s2_ref[...] += ps2


def _layer_body(h_ref, a0_ref, a1_ref, bt_ref, Wa_ref, ba_ref, Wb_ref, bb_ref,
                ms_ref, wt_ref, bs_ref, o_ref, ts_ref, s0_ref, s1_ref, s2_ref):
    i = pl.program_id(0)
    j = i % _GRID

    @pl.when(i < _GRID)
    def _():
        _mlp_stats_phase(h_ref, a0_ref, a1_ref, bt_ref, Wa_ref, ba_ref, Wb_ref,
                         bb_ref, ts_ref, s0_ref, s1_ref, s2_ref, i, j)

    @pl.when(i >= _GRID)
    def _():
        o_ref[...] = _norm_from_stats(ts_ref, j, bt_ref, ms_ref, wt_ref, bs_ref,
                                      s0_ref, s1_ref, s2_ref)


def _w1(i):
    return (jnp.where(i < _GRID, i, 0), 0)


def _w2(i):
    return (jnp.where(i < _GRID, 0, i % _GRID), 0)


def _jj(i):
    return (i % _GRID, 0)


def _jj3(i):
    return (i % _GRID, 0, 0)


def _c2(i):
    return (0, 0)


_WSPEC = pl.BlockSpec((D, H), _c2)
_BSPEC = pl.BlockSpec((1, H), _c2)


def _layer_tc(h, agg2, batch3, Wa, ba, Wb, bb, ms, wt, bs):
    return pl.pallas_call(
        _layer_body,
        grid=(2 * _GRID,),
        in_specs=[
            pl.BlockSpec((_BLK, H), _w1),
            pl.BlockSpec((1, _BLK, D), lambda i: (0, jnp.where(i < _GRID, i, 0), 0)),
            pl.BlockSpec((1, _BLK, D), lambda i: (1, jnp.where(i < _GRID, i, 0), 0)),
            pl.BlockSpec((1, 1, _BLK), _jj3),
            _WSPEC, _BSPEC, _WSPEC, _BSPEC,
            _BSPEC, _BSPEC, _BSPEC,
        ],
        out_specs=pl.BlockSpec((_BLK, H), _w2),
        out_shape=jax.ShapeDtypeStruct((N, H), jnp.float32),
        scratch_shapes=[
            pltpu.VMEM((N, H), jnp.float32),
            pltpu.VMEM((B, H), jnp.float32),
            pltpu.VMEM((B, H), jnp.float32),
            pltpu.VMEM((B, H), jnp.float32),
        ],
    )(h, agg2, agg2, batch3, Wa, ba, Wb, bb, ms, wt, bs)


def _layer3_pool_body(h_ref, a0_ref, a1_ref, bt_ref, Wa_ref, ba_ref, Wb_ref,
                      bb_ref, ms_ref, wt_ref, bs_ref, h1_ref, Wg_ref,
                      bg_ref, ho_ref, s0o_ref, gmax_ref, hmax_ref, sh_ref,
                      ts_ref, s0_ref, s1_ref, s2_ref):
    i = pl.program_id(0)
    j = i % _GRID

    @pl.when(i < _GRID)
    def _():
        _mlp_stats_phase(h_ref, a0_ref, a1_ref, bt_ref, Wa_ref, ba_ref, Wb_ref,
                         bb_ref, ts_ref, s0_ref, s1_ref, s2_ref, i, j)

    @pl.when(i >= _GRID)
    def _():
        h3 = _norm_from_stats(ts_ref, j, bt_ref, ms_ref, wt_ref, bs_ref,
                              s0_ref, s1_ref, s2_ref)
        h = h1_ref[...] + h_ref[...] + h3
        ho_ref[...] = h
        bvec = bt_ref[0, 0, :]
        PT = _pt(bvec)
        sh = _hdot(PT, h)
        gate = _xdot(h, Wg_ref[...]) + bg_ref[...]
        M = (lax.broadcasted_iota(jnp.int32, (_BLK, B), 1)
             == bvec[:, None]).astype(jnp.float32)
        G = gate + (M - 1.0) * 1e30
        gm = jnp.broadcast_to(jnp.max(G, axis=0)[:, None], (B, H))
        rows = []
        for b in range(B):
            mb = (bvec[:, None] == b).astype(jnp.float32)
            rows.append(jnp.max(h * mb, axis=0))
        hm = jnp.stack(rows)

        @pl.when(i == _GRID)
        def _():
            s0o_ref[...] = s0_ref[...]
            gmax_ref[...] = gm
            hmax_ref[...] = hm
            sh_ref[...] = sh

        @pl.when(i > _GRID)
        def _():
            gmax_ref[...] = jnp.maximum(gmax_ref[...], gm)
            hmax_ref[...] = jnp.maximum(hmax_ref[...], hm)
            sh_ref[...] += sh


def _layer3_pool(h, agg2, batch3, Wa, ba, Wb, bb, ms, wt, bs, h1, Wg, bg):
    bh = jax.ShapeDtypeStruct((B, H), jnp.float32)
    return pl.pallas_call(
        _layer3_pool_body,
        grid=(2 * _GRID,),
        in_specs=[
            pl.BlockSpec((_BLK, H), _jj),
            pl.BlockSpec((1, _BLK, D), lambda i: (0, jnp.where(i < _GRID, i, 0), 0)),
            pl.BlockSpec((1, _BLK, D), lambda i: (1, jnp.where(i < _GRID, i, 0), 0)),
            pl.BlockSpec((1, 1, _BLK), _jj3),
            _WSPEC, _BSPEC, _WSPEC, _BSPEC,
            _BSPEC, _BSPEC, _BSPEC,
            pl.BlockSpec((_BLK, H), _w2),
            pl.BlockSpec((H, 1), _c2),
            pl.BlockSpec((1, 1), _c2),
        ],
        out_specs=[
            pl.BlockSpec((_BLK, H), _w2),
            pl.BlockSpec((B, H), _c2),
            pl.BlockSpec((B, H), _c2),
            pl.BlockSpec((B, H), _c2),
            pl.BlockSpec((B, H), _c2),
        ],
        out_shape=[jax.ShapeDtypeStruct((N, H), jnp.float32), bh, bh, bh, bh],
        scratch_shapes=[
            pltpu.VMEM((N, H), jnp.float32),
            pltpu.VMEM((B, H), jnp.float32),
            pltpu.VMEM((B, H), jnp.float32),
            pltpu.VMEM((B, H), jnp.float32),
        ],
    )(h, agg2, agg2, batch3, Wa, ba, Wb, bb, ms, wt, bs, h1, Wg, bg)


def _pool2_head_body(h_ref, bt_ref, Wg_ref, bg_ref, gmax_ref, s0_ref, sh_ref,
                     hmax_ref, up_ref, Wc1_ref, bc1_ref, Wc2_ref, bc2_ref,
                     o_ref, sex_ref, shex_ref):
    i = pl.program_id(0)

    @pl.when(i < _GRID)
    def _():
        h = h_ref[...]
        gate = _xdot(h, Wg_ref[...]) + bg_ref[...]
        bvec = bt_ref[0, 0, :]
        PT = _pt(bvec)
        gmax_col = gmax_ref[...][:, 0:1]
        gmax_rows = lax.dot_general(PT, gmax_col, _DN0,
                                    preferred_element_type=jnp.float32,
                                    precision=_HI)
        ex = jnp.exp(gate - gmax_rows)
        psex = _hdot(PT, jnp.broadcast_to(ex, (_BLK, H)))
        pshex = _hdot(PT, h * ex)

        @pl.when(i == 0)
        def _():
            sex_ref[...] = psex
            shex_ref[...] = pshex

        @pl.when(i > 0)
        def _():
            sex_ref[...] += psex
            shex_ref[...] += pshex

    @pl.when(i == _GRID)
    def _():
        cnt = jnp.maximum(s0_ref[...], 1.0)
        att = shex_ref[...] / jnp.maximum(sex_ref[...], 1e-30)
        meanp = sh_ref[...] / cnt
        z = jnp.concatenate([att, meanp, hmax_ref[...], up_ref[...]], axis=1)
        z1 = jnp.maximum(_xdot(z, Wc1_ref[...]) + bc1_ref[...], 0.0)
        o_ref[...] = _xdot(z1, Wc2_ref[...]) + bc2_ref[...]


def _pool2_head(h, batch3, Wg, bg, gmax, s0, sh, hmax, up, Wc1p, bc1, Wc2, bc2):
    bhspec = pl.BlockSpec((B, H), _c2)
    return pl.pallas_call(
        _pool2_head_body,
        grid=(_GRID + 1,),
        in_specs=[
            pl.BlockSpec((_BLK, H), lambda i: (jnp.where(i < _GRID, i, 0), 0)),
            pl.BlockSpec((1, 1, _BLK), lambda i: (jnp.where(i < _GRID, i, 0), 0, 0)),
            pl.BlockSpec((H, 1), _c2),
            pl.BlockSpec((1, 1), _c2),
            bhspec, bhspec, bhspec, bhspec,
            pl.BlockSpec((B, H), _c2),
            pl.BlockSpec((4 * H, H), _c2),
            _BSPEC,
            pl.BlockSpec((H, 1), _c2),
            pl.BlockSpec((1, 1), _c2),
        ],
        out_specs=pl.BlockSpec((B, 1), _c2),
        out_shape=jax.ShapeDtypeStruct((B, 1), jnp.float32),
        scratch_shapes=[
            pltpu.VMEM((B, H), jnp.float32),
            pltpu.VMEM((B, H), jnp.float32),
        ],
    )(h, batch3, Wg, bg, gmax, s0, sh, hmax, up, Wc1p, bc1, Wc2, bc2)


def kernel(x, edge_index, batch, u,
           W1a, b1a, W1b, b1b, gn1_w, gn1_b, gn1_ms,
           W2a, b2a, W2b, b2b, gn2_w, gn2_b, gn2_ms,
           W3a, b3a, W3b, b3b, gn3_w, gn3_b, gn3_ms,
           Wg, bg, Wc1, bc1, Wc2, bc2):
    ei = jnp.stack([edge_index[0].reshape(_NW, _NIT, _CHUNK),
                    edge_index[1].reshape(_NW, _NIT, _CHUNK)], axis=2)
    zeros = jnp.zeros((_NPAD, D), jnp.float32)
    batch3 = batch.reshape(_GRID, 1, _BLK)
    r1 = lambda v: v.reshape(1, -1)

    agg1 = _agg_sc(x, ei, zeros)
    h1 = _layer_tc(x, agg1, batch3, W1a, r1(b1a), W1b, r1(b1b),
                   r1(gn1_ms), r1(gn1_w), r1(gn1_b))
    agg2 = _agg_sc(h1, ei, zeros)
    h2 = _layer_tc(h1, agg2, batch3, W2a, r1(b2a), W2b, r1(b2b),
                   r1(gn2_ms), r1(gn2_w), r1(gn2_b))
    agg3 = _agg_sc(h2, ei, zeros)
    hsum, s0, gmax, hmax, sh = _layer3_pool(
        h2, agg3, batch3, W3a, r1(b3a), W3b, r1(b3b),
        r1(gn3_ms), r1(gn3_w), r1(gn3_b), h1, Wg, bg.reshape(1, 1))

    up = jnp.concatenate([u, jnp.zeros((B, H - 3), jnp.float32)], axis=1)
    Wc1p = jnp.concatenate([Wc1, jnp.zeros((4 * H - (3 * H + 3), H), jnp.float32)],
                           axis=0)
    out = _pool2_head(hsum, batch3, Wg, bg.reshape(1, 1), gmax, s0, sh, hmax,
                      up, Wc1p, r1(bc1), Wc2, bc2.reshape(1, 1))
    return out[:, 0]
